# baseline (speedup 1.0000x reference)
import functools

import jax
import jax.numpy as jnp
from jax import lax
from jax.experimental import pallas as pl
from jax.experimental.pallas import tpu as pltpu
from jax.experimental.pallas import tpu_sc as plsc

HEADS = 4
OUT_F = 32
F1 = HEADS * OUT_F
NC = 2
NS = 16
NW = NC * NS
L = 16
K = 128

f32 = jnp.float32
i32 = jnp.int32


def _take16(v, idx):
  return jnp.take_along_axis(v, idx, axis=0, mode="promise_in_bounds")


def _splat16(v, i):
  return _take16(v, jnp.full((L,), i, dtype=i32))


def _store_tables(feat, fc, Al, Ar, T_ref, elr_ref):
  h = jnp.dot(feat, fc, preferred_element_type=f32)
  n = h.shape[0]
  np_ = T_ref.shape[0]
  T_ref[...] = jnp.concatenate([h, jnp.zeros((np_ - n, F1), f32)], axis=0)
  cdims = (((0,), (1,)), ((), ()))
  el_t = lax.dot_general(Al, h, cdims, preferred_element_type=f32)
  er_t = lax.dot_general(Ar, h, cdims, preferred_element_type=f32)
  elr = jnp.concatenate([el_t, er_t], axis=0)
  elr_ref[...] = jnp.concatenate(
      [elr, jnp.zeros((2 * HEADS, np_ - n), f32)], axis=1)


def _tc_prep_body(doc_ref, word_ref, Wd_ref, bd_ref, Ww_ref, bw_ref,
                  fc_ref, Al_ref, Ar_ref, res_ref,
                  T_ref, elr_ref, resid_ref):
  d = jnp.maximum(jnp.dot(doc_ref[...], Wd_ref[...],
                          preferred_element_type=f32) + bd_ref[...], 0.0)
  w = jnp.maximum(jnp.dot(word_ref[...], Ww_ref[...],
                          preferred_element_type=f32) + bw_ref[...], 0.0)
  feat = jnp.concatenate([d, w], axis=0)
  _store_tables(feat, fc_ref[...], Al_ref[...], Ar_ref[...], T_ref, elr_ref)
  resid_ref[...] = jnp.dot(feat, res_ref[...], preferred_element_type=f32)


def _combine(msg_cat, den, resid, b):
  n = resid.shape[0]
  msg = msg_cat[0, :n] + msg_cat[1, :n]
  dsum = (den[0] + den[1])[:, :n]
  sel = (lax.broadcasted_iota(i32, (HEADS, F1), 1) // OUT_F
         == lax.broadcasted_iota(i32, (HEADS, F1), 0)).astype(f32)
  denf = lax.dot_general(dsum, sel, (((0,), (0,)), ((), ())),
                         preferred_element_type=f32)
  return jnp.maximum(msg / (denf + 1e-9) + resid + b[None, :], 0.0)


def _tc_mid_body(msg_ref, den_ref, resid_ref, b_ref, fc_ref, Al_ref,
                 Ar_ref, T_ref, elr_ref, feat_ref):
  feat = _combine(msg_ref[...], den_ref[...], resid_ref[...], b_ref[...])
  feat_ref[...] = feat
  _store_tables(feat, fc_ref[...], Al_ref[...], Ar_ref[...], T_ref, elr_ref)


def _tc_final_body(msg_ref, den_ref, resid_ref, b_ref, W2_ref,
                   b2_ref, Wc_ref, bc_ref, out_ref):
  feat = _combine(msg_ref[...], den_ref[...], resid_ref[...], b_ref[...])
  g = jnp.max(feat, axis=0, keepdims=True)
  h2 = jnp.maximum(jnp.dot(g, W2_ref[...], preferred_element_type=f32)
                   + b2_ref[...], 0.0)
  out_ref[...] = jnp.dot(h2, Wc_ref[...], preferred_element_type=f32) \
      + bc_ref[...]


def _make_edge_kernel(n_pad, e_real, e0b, ch):
  rows_per_tile = n_pad // NS
  den_rows = n_pad // L
  drows_per_tile = den_rows // NS
  drain = []
  off = 0
  while off < rows_per_tile:
    sz = min(K, rows_per_tile - off)
    drain.append((off, sz))
    off += sz

  mesh = plsc.VectorSubcoreMesh(core_axis_name="c", subcore_axis_name="s")

  @functools.partial(
      pl.kernel,
      out_type=[
          jax.ShapeDtypeStruct((NC, n_pad, OUT_F), f32),
          jax.ShapeDtypeStruct((NC, den_rows, L), f32),
      ],
      mesh=mesh,
      scratch_types=[
          pltpu.VMEM((n_pad, 2), f32),
      ] + [pltpu.VMEM((K,), i32)] * 4
      + [pltpu.VMEM((K,), i32)] * 4
      + [pltpu.VMEM((K, OUT_F), f32)] * 2
      + [pltpu.VMEM((K, OUT_F), f32)] * 2
      + [pltpu.VMEM((K, L), f32)] * 2
      + [
          pltpu.VMEM((K, OUT_F), f32),
          pltpu.VMEM_SHARED((n_pad, OUT_F), f32),
          pltpu.VMEM_SHARED((den_rows, L), f32),
      ] + [pltpu.SemaphoreType.DMA] * 6,
      compiler_params=pltpu.CompilerParams(use_tc_tiling_on_sc=False,
                                          needs_layout_passes=False),
  )
  def edge_kernel(src_hbm, dst_hbm, t_hbm, elr_hbm, msg_hbm, den_hbm,
                  elr_v, s0, s1, d0, d1,
                  e0_, e1_, q0, q1,
                  r0_, r1_, o0, o1, n0, n1,
                  zero_v, acc_sh, accd_sh,
                  g0, g1, i0, i1, c0, c1):
    srcs, dsts = (s0, s1), (d0, d1)
    sdst, sdstq = (e0_, e1_), (q0, q1)
    rows, outbs, dens = (r0_, r1_), (o0, o1), (n0, n1)
    gsems, isems, ssems = (g0, g1), (i0, i1), (c0, c1)
    c = lax.axis_index("c")
    s = lax.axis_index("s")
    wid = c * NS + s
    base0 = wid * (ch * K)
    row0 = s * rows_per_tile
    drow0 = s * drows_per_tile

    pltpu.sync_copy(elr_hbm, elr_v)

    zeros16 = jnp.zeros((L,), f32)
    zeros16i = jnp.zeros((L,), i32)

    def zrow(r, _):
      for v in range(OUT_F // L):
        zero_v[r, pl.ds(v * L, L)] = zeros16
      dens[0][r, pl.ds(0, L)] = zeros16
      return 0

    lax.fori_loop(0, K, zrow, 0)
    for off_, sz in drain:
      pltpu.sync_copy(zero_v.at[pl.ds(0, sz)],
                      acc_sh.at[pl.ds(row0 + off_, sz)])
    pltpu.sync_copy(dens[0].at[pl.ds(0, drows_per_tile)],
                    accd_sh.at[pl.ds(drow0, drows_per_tile)])
    plsc.subcore_barrier()

    iota16 = lax.iota(i32, L)
    czero = jnp.zeros((L,), i32)
    cone = jnp.full((L,), 1, i32)

    def issue_idx(b, kq):
      eb = base0 + kq * K
      pltpu.async_copy(src_hbm.at[pl.ds(eb, K)], srcs[b], isems[b])
      pltpu.async_copy(dst_hbm.at[pl.ds(eb, K)], dsts[b], isems[b])

    def wait_idx(b):
      pltpu.make_async_copy(src_hbm.at[pl.ds(0, K)], srcs[b], isems[b]).wait()
      pltpu.make_async_copy(dst_hbm.at[pl.ds(0, K)], dsts[b], isems[b]).wait()

    def issue_gather(b):
      pltpu.async_copy(t_hbm.at[srcs[b]], rows[b], gsems[b])

    def wait_gather(b):
      pltpu.make_async_copy(t_hbm.at[srcs[b]], rows[b], gsems[b]).wait()

    def issue_scatter(b):
      pltpu.async_copy(outbs[b], acc_sh.at[sdst[b]], ssems[b], add=True)
      pltpu.async_copy(dens[b], accd_sh.at[sdstq[b]], ssems[b], add=True)

    def wait_scatter(b):
      pltpu.make_async_copy(outbs[b], acc_sh.at[sdst[b]], ssems[b]).wait()
      pltpu.make_async_copy(dens[b], accd_sh.at[sdstq[b]], ssems[b]).wait()

    def compute(b, eb):
      sb = b
      src_v, dst_v, rows_v = srcs[b], dsts[b], rows[b]
      outb_v, den_v = outbs[sb], dens[sb]

      def group_body(g, _):
        r0 = g * L
        src16 = src_v[pl.ds(r0, L)]
        dst16 = dst_v[pl.ds(r0, L)]
        ids = jnp.full((L,), eb, dtype=i32) + r0 + iota16
        keep = (src16 != dst16) | ((ids >= e0b) & (ids < e_real))
        mf = jnp.where(keep, 1.0, 0.0).astype(f32)
        el16 = plsc.load_gather(elr_v, [src16, czero])
        er16 = plsc.load_gather(elr_v, [dst16, cone])
        e16 = el16 + er16
        e16 = jnp.maximum(e16, 0.2 * e16)
        w16 = jnp.exp(e16) * mf
        sdst[sb][pl.ds(r0, L)] = dst16
        sdstq[sb][pl.ds(r0, L)] = lax.shift_right_logical(dst16, 4)
        dm16 = dst16 & (L - 1)
        for i in range(L):
          r = r0 + i
          ws = _splat16(w16, i)
          dmi = _splat16(dm16, i)
          den_v[r, pl.ds(0, L)] = jnp.where(iota16 == dmi, ws, 0.0)
          for half in range(OUT_F // L):
            col = half * L
            outb_v[r, pl.ds(col, L)] = ws * rows_v[r, pl.ds(col, L)]
        return 0

      lax.fori_loop(0, K // L, group_body, 0)

    issue_idx(0, 0)
    issue_idx(1, 1)
    wait_idx(0)
    issue_gather(0)

    def super_body(k2, _):
      for b in range(2):
        k = k2 * 2 + b
        wait_idx(1 - b)
        issue_gather(1 - b)
        wait_gather(b)

        @pl.when(k2 > 0)
        def _():
          wait_scatter(b)

        compute(b, base0 + k * K)
        issue_scatter(b)
        issue_idx(b, jnp.minimum(k + 2, ch - 1))
      return 0

    lax.fori_loop(0, ch // 2, super_body, 0)
    for b in range(2):
      wait_scatter(b)
    wait_gather(0)
    wait_idx(1)
    plsc.subcore_barrier()

    for off_, sz in drain:
      pltpu.sync_copy(acc_sh.at[pl.ds(row0 + off_, sz)],
                      msg_hbm.at[c, pl.ds(row0 + off_, sz)])
    pltpu.sync_copy(accd_sh.at[pl.ds(drow0, drows_per_tile)],
                    den_hbm.at[c, pl.ds(drow0, drows_per_tile)])

  return edge_kernel


def _attn_mat(a):
  flat = a.reshape(F1)
  hd = jnp.arange(F1, dtype=i32)[:, None] // OUT_F
  return jnp.where(hd == jnp.arange(HEADS, dtype=i32)[None, :],
                   flat[:, None], 0.0)


def kernel(doc_feat, word_feat, edge_index, Wd, bd, Ww, bw, fc0, al0, ar0,
           b0, res0, fc1, al1, ar1, b1, W2, b2, Wc, bc):
  nd = doc_feat.shape[0]
  nw = word_feat.shape[0]
  n = nd + nw
  e0 = edge_index.shape[1]
  e0b = 2 * e0
  e_real = e0b + n
  chk = NW * K
  ch = 2 * (-(-e_real // (2 * chk)))
  e_pad = ch * chk
  n_pad = NS * L * (-(-n // (NS * L)))

  ei = edge_index.astype(i32)
  loop = jnp.arange(n, dtype=i32)
  pad = jnp.zeros((e_pad - e_real,), i32)
  src = jnp.concatenate([ei[0], ei[1], loop, pad])
  dst = jnp.concatenate([ei[1], ei[0], loop, pad])

  al0m, ar0m = _attn_mat(al0), _attn_mat(ar0)
  al1m, ar1m = _attn_mat(al1), _attn_mat(ar1)

  table_shapes = [
      jax.ShapeDtypeStruct((n_pad, F1), f32),
      jax.ShapeDtypeStruct((2 * HEADS, n_pad), f32),
  ]

  def run_layer(edge_kernel, T, elr_t):
    msgs, dens = [], []
    for hd in range(HEADS):
      th = lax.slice_in_dim(T, hd * OUT_F, (hd + 1) * OUT_F, axis=1)
      eh = jnp.stack([elr_t[hd], elr_t[HEADS + hd]], axis=1)
      m, d = edge_kernel(src, dst, th, eh)
      msgs.append(m)
      dens.append(d)
    msg_cat = jnp.concatenate(msgs, axis=2)
    den = jnp.stack([d.reshape(NC, n_pad) for d in dens], axis=1)
    return msg_cat, den

  T0, elr0, resid0 = pl.pallas_call(
      _tc_prep_body,
      out_shape=table_shapes + [jax.ShapeDtypeStruct((n, F1), f32)],
  )(doc_feat, word_feat, Wd, bd, Ww, bw, fc0, al0m, ar0m, res0)

  edge_kernel = _make_edge_kernel(n_pad, e_real, e0b, ch)
  msg0, den0 = run_layer(edge_kernel, T0, elr0)

  T1, elr1, feat1 = pl.pallas_call(
      _tc_mid_body,
      out_shape=table_shapes + [jax.ShapeDtypeStruct((n, F1), f32)],
  )(msg0, den0, resid0, b0, fc1, al1m, ar1m)

  msg1, den1 = run_layer(edge_kernel, T1, elr1)

  out = pl.pallas_call(
      _tc_final_body,
      out_shape=jax.ShapeDtypeStruct((1, 2), f32),
  )(msg1, den1, feat1, b1, W2, b2, Wc, bc)

  return out


# speedup vs baseline: 1.1034x; 1.1034x over previous
import functools

import jax
import jax.numpy as jnp
from jax import lax
from jax.experimental import pallas as pl
from jax.experimental.pallas import tpu as pltpu
from jax.experimental.pallas import tpu_sc as plsc

HEADS = 4
OUT_F = 32
F1 = HEADS * OUT_F
NC = 2
NS = 16
NW = NC * NS
L = 16
K = 128

f32 = jnp.float32
i32 = jnp.int32


def _take16(v, idx):
  return jnp.take_along_axis(v, idx, axis=0, mode="promise_in_bounds")


def _splat16(v, i):
  return _take16(v, jnp.full((L,), i, dtype=i32))


def _store_tables(feat, fc, Al, Ar, T_ref, elr_ref):
  h = jnp.dot(feat, fc, preferred_element_type=f32)
  n = h.shape[0]
  np_ = T_ref.shape[0]
  T_ref[...] = jnp.concatenate([h, jnp.zeros((np_ - n, F1), f32)], axis=0)
  cdims = (((0,), (1,)), ((), ()))
  el_t = lax.dot_general(Al, h, cdims, preferred_element_type=f32)
  er_t = lax.dot_general(Ar, h, cdims, preferred_element_type=f32)
  elr = jnp.concatenate([el_t, er_t], axis=0)
  elr_ref[...] = jnp.concatenate(
      [elr, jnp.zeros((2 * HEADS, np_ - n), f32)], axis=1)


def _tc_prep_body(doc_ref, word_ref, Wd_ref, bd_ref, Ww_ref, bw_ref,
                  fc_ref, Al_ref, Ar_ref, res_ref,
                  T_ref, elr_ref, resid_ref):
  d = jnp.maximum(jnp.dot(doc_ref[...], Wd_ref[...],
                          preferred_element_type=f32) + bd_ref[...], 0.0)
  w = jnp.maximum(jnp.dot(word_ref[...], Ww_ref[...],
                          preferred_element_type=f32) + bw_ref[...], 0.0)
  feat = jnp.concatenate([d, w], axis=0)
  _store_tables(feat, fc_ref[...], Al_ref[...], Ar_ref[...], T_ref, elr_ref)
  resid_ref[...] = jnp.dot(feat, res_ref[...], preferred_element_type=f32)


def _combine(msg_cat, den, resid, b):
  n = resid.shape[0]
  msg = msg_cat[0, :n] + msg_cat[1, :n]
  dsum = (den[0] + den[1])[:, :n]
  sel = (lax.broadcasted_iota(i32, (HEADS, F1), 1) // OUT_F
         == lax.broadcasted_iota(i32, (HEADS, F1), 0)).astype(f32)
  denf = lax.dot_general(dsum, sel, (((0,), (0,)), ((), ())),
                         preferred_element_type=f32)
  return jnp.maximum(msg / (denf + 1e-9) + resid + b[None, :], 0.0)


def _tc_mid_body(msg_ref, den_ref, resid_ref, b_ref, fc_ref, Al_ref,
                 Ar_ref, T_ref, elr_ref, feat_ref):
  feat = _combine(msg_ref[...], den_ref[...], resid_ref[...], b_ref[...])
  feat_ref[...] = feat
  _store_tables(feat, fc_ref[...], Al_ref[...], Ar_ref[...], T_ref, elr_ref)


def _tc_final_body(msg_ref, den_ref, resid_ref, b_ref, W2_ref,
                   b2_ref, Wc_ref, bc_ref, out_ref):
  feat = _combine(msg_ref[...], den_ref[...], resid_ref[...], b_ref[...])
  g = jnp.max(feat, axis=0, keepdims=True)
  h2 = jnp.maximum(jnp.dot(g, W2_ref[...], preferred_element_type=f32)
                   + b2_ref[...], 0.0)
  out_ref[...] = jnp.dot(h2, Wc_ref[...], preferred_element_type=f32) \
      + bc_ref[...]


def _make_edge_kernel(n_pad, e_real, e0b, ch):
  rows_per_tile = n_pad // NS
  den_rows = n_pad // L
  drows_per_tile = den_rows // NS
  drain = []
  off = 0
  while off < rows_per_tile:
    sz = min(K, rows_per_tile - off)
    drain.append((off, sz))
    off += sz

  mesh = plsc.VectorSubcoreMesh(core_axis_name="c", subcore_axis_name="s")

  @functools.partial(
      pl.kernel,
      out_type=[
          jax.ShapeDtypeStruct((NC, n_pad, OUT_F), f32),
          jax.ShapeDtypeStruct((NC, den_rows, L), f32),
      ],
      mesh=mesh,
      scratch_types=[
          pltpu.VMEM((n_pad, 2), f32),
      ] + [pltpu.VMEM((K,), i32)] * 8
      + [
          pltpu.VMEM((K,), i32),
      ] + [pltpu.VMEM((K, OUT_F), f32)] * 4
      + [
          pltpu.VMEM((K, OUT_F), f32),
          pltpu.VMEM((K, L), f32),
          pltpu.VMEM((K, OUT_F), f32),
          pltpu.VMEM_SHARED((n_pad, OUT_F), f32),
          pltpu.VMEM_SHARED((den_rows, L), f32),
      ] + [pltpu.SemaphoreType.DMA] * 8,
      compiler_params=pltpu.CompilerParams(use_tc_tiling_on_sc=False,
                                           needs_layout_passes=False),
  )
  def edge_kernel(src_hbm, dst_hbm, t_hbm, elr_hbm, msg_hbm, den_hbm,
                  elr_v, s0, s1, s2, s3, d0, d1, d2, d3, dstq_v,
                  r0_, r1_, r2_, r3_, outb_v, den_v,
                  zero_v, acc_sh, accd_sh, g0, g1, g2, g3, i0, i1, i2, i3):
    srcs, dsts = (s0, s1, s2, s3), (d0, d1, d2, d3)
    rows = (r0_, r1_, r2_, r3_)
    gsems, isems = (g0, g1, g2, g3), (i0, i1, i2, i3)
    c = lax.axis_index("c")
    s = lax.axis_index("s")
    wid = c * NS + s
    base0 = wid * (ch * K)
    row0 = s * rows_per_tile
    drow0 = s * drows_per_tile

    pltpu.sync_copy(elr_hbm, elr_v)

    zeros16 = jnp.zeros((L,), f32)

    def zrow(r, _):
      for v in range(OUT_F // L):
        zero_v[r, pl.ds(v * L, L)] = zeros16
      den_v[r, pl.ds(0, L)] = zeros16
      return 0

    lax.fori_loop(0, K, zrow, 0)
    for off_, sz in drain:
      pltpu.sync_copy(zero_v.at[pl.ds(0, sz)],
                      acc_sh.at[pl.ds(row0 + off_, sz)])
    pltpu.sync_copy(den_v.at[pl.ds(0, drows_per_tile)],
                    accd_sh.at[pl.ds(drow0, drows_per_tile)])
    plsc.subcore_barrier()

    iota16 = lax.iota(i32, L)
    czero = jnp.zeros((L,), i32)
    cone = jnp.full((L,), 1, i32)

    def issue_idx(b, kq):
      eb = base0 + kq * K
      pltpu.async_copy(src_hbm.at[pl.ds(eb, K)], srcs[b], isems[b])
      pltpu.async_copy(dst_hbm.at[pl.ds(eb, K)], dsts[b], isems[b])

    def wait_idx(b):
      pltpu.make_async_copy(src_hbm.at[pl.ds(0, K)], srcs[b], isems[b]).wait()
      pltpu.make_async_copy(dst_hbm.at[pl.ds(0, K)], dsts[b], isems[b]).wait()

    def issue_gather(b):
      pltpu.async_copy(t_hbm.at[srcs[b]], rows[b], gsems[b])

    def wait_gather(b):
      pltpu.make_async_copy(t_hbm.at[srcs[b]], rows[b], gsems[b]).wait()

    def compute(b, eb):
      src_v, dst_v, rows_v = srcs[b], dsts[b], rows[b]

      def group_body(g, _):
        r0 = g * L
        src16 = src_v[pl.ds(r0, L)]
        dst16 = dst_v[pl.ds(r0, L)]
        ids = jnp.full((L,), eb, dtype=i32) + r0 + iota16
        keep = (src16 != dst16) | ((ids >= e0b) & (ids < e_real))
        mf = jnp.where(keep, 1.0, 0.0).astype(f32)
        el16 = plsc.load_gather(elr_v, [src16, czero])
        er16 = plsc.load_gather(elr_v, [dst16, cone])
        e16 = el16 + er16
        e16 = jnp.maximum(e16, 0.2 * e16)
        w16 = jnp.exp(e16) * mf
        dstq_v[pl.ds(r0, L)] = lax.shift_right_logical(dst16, 4)
        for i in range(L):
          den_v[r0 + i, pl.ds(0, L)] = zeros16
        plsc.store_scatter(den_v, [r0 + iota16, dst16 & (L - 1)], w16)
        for i in range(L):
          r = r0 + i
          ws = _splat16(w16, i)
          for half in range(OUT_F // L):
            col = half * L
            outb_v[r, pl.ds(col, L)] = ws * rows_v[r, pl.ds(col, L)]
        return 0

      lax.fori_loop(0, K // L, group_body, 0)
      pltpu.sync_copy(outb_v, acc_sh.at[dst_v], add=True)
      pltpu.sync_copy(den_v, accd_sh.at[dstq_v], add=True)

    for b in range(4):
      issue_idx(b, b)
    for b in range(2):
      wait_idx(b)
      issue_gather(b)

    def super_body(k4, _):
      for b in range(4):
        k = k4 * 4 + b
        wait_idx((b + 2) % 4)
        issue_gather((b + 2) % 4)
        wait_gather(b)
        compute(b, base0 + k * K)
        issue_idx(b, jnp.minimum(k + 4, ch - 1))
      return 0

    lax.fori_loop(0, ch // 4, super_body, 0)
    for b in (0, 1):
      wait_gather(b)
    for b in (2, 3):
      wait_idx(b)
    plsc.subcore_barrier()

    for off_, sz in drain:
      pltpu.sync_copy(acc_sh.at[pl.ds(row0 + off_, sz)],
                      msg_hbm.at[c, pl.ds(row0 + off_, sz)])
    pltpu.sync_copy(accd_sh.at[pl.ds(drow0, drows_per_tile)],
                    den_hbm.at[c, pl.ds(drow0, drows_per_tile)])

  return edge_kernel


def _attn_mat(a):
  flat = a.reshape(F1)
  hd = jnp.arange(F1, dtype=i32)[:, None] // OUT_F
  return jnp.where(hd == jnp.arange(HEADS, dtype=i32)[None, :],
                   flat[:, None], 0.0)


def kernel(doc_feat, word_feat, edge_index, Wd, bd, Ww, bw, fc0, al0, ar0,
           b0, res0, fc1, al1, ar1, b1, W2, b2, Wc, bc):
  nd = doc_feat.shape[0]
  nw = word_feat.shape[0]
  n = nd + nw
  e0 = edge_index.shape[1]
  e0b = 2 * e0
  e_real = e0b + n
  chk = NW * K
  ch = 4 * (-(-e_real // (4 * chk)))
  e_pad = ch * chk
  n_pad = NS * L * (-(-n // (NS * L)))

  ei = edge_index.astype(i32)
  loop = jnp.arange(n, dtype=i32)
  pad = jnp.zeros((e_pad - e_real,), i32)
  src = jnp.concatenate([ei[0], ei[1], loop, pad])
  dst = jnp.concatenate([ei[1], ei[0], loop, pad])

  al0m, ar0m = _attn_mat(al0), _attn_mat(ar0)
  al1m, ar1m = _attn_mat(al1), _attn_mat(ar1)

  table_shapes = [
      jax.ShapeDtypeStruct((n_pad, F1), f32),
      jax.ShapeDtypeStruct((2 * HEADS, n_pad), f32),
  ]

  def run_layer(edge_kernel, T, elr_t):
    msgs, dens = [], []
    for hd in range(HEADS):
      th = lax.slice_in_dim(T, hd * OUT_F, (hd + 1) * OUT_F, axis=1)
      eh = jnp.stack([elr_t[hd], elr_t[HEADS + hd]], axis=1)
      m, d = edge_kernel(src, dst, th, eh)
      msgs.append(m)
      dens.append(d)
    msg_cat = jnp.concatenate(msgs, axis=2)
    den = jnp.stack([d.reshape(NC, n_pad) for d in dens], axis=1)
    return msg_cat, den

  T0, elr0, resid0 = pl.pallas_call(
      _tc_prep_body,
      out_shape=table_shapes + [jax.ShapeDtypeStruct((n, F1), f32)],
  )(doc_feat, word_feat, Wd, bd, Ww, bw, fc0, al0m, ar0m, res0)

  edge_kernel = _make_edge_kernel(n_pad, e_real, e0b, ch)
  msg0, den0 = run_layer(edge_kernel, T0, elr0)

  T1, elr1, feat1 = pl.pallas_call(
      _tc_mid_body,
      out_shape=table_shapes + [jax.ShapeDtypeStruct((n, F1), f32)],
  )(msg0, den0, resid0, b0, fc1, al1m, ar1m)

  msg1, den1 = run_layer(edge_kernel, T1, elr1)

  out = pl.pallas_call(
      _tc_final_body,
      out_shape=jax.ShapeDtypeStruct((1, 2), f32),
  )(msg1, den1, feat1, b1, W2, b2, Wc, bc)

  return out


# speedup vs baseline: 1.5051x; 1.3641x over previous
import functools

import jax
import jax.numpy as jnp
from jax import lax
from jax.experimental import pallas as pl
from jax.experimental.pallas import tpu as pltpu
from jax.experimental.pallas import tpu_sc as plsc

HEADS = 4
OUT_F = 32
F1 = HEADS * OUT_F
NC = 2
NS = 16
NW = NC * NS
L = 16
K = 128

f32 = jnp.float32
i32 = jnp.int32


def _take16(v, idx):
  return jnp.take_along_axis(v, idx, axis=0, mode="promise_in_bounds")


def _splat16(v, i):
  return _take16(v, jnp.full((L,), i, dtype=i32))


def _store_tables(feat, fc, Al, Ar, T_ref, elr_ref):
  h = jnp.dot(feat, fc, preferred_element_type=f32)
  n = h.shape[0]
  np_ = T_ref.shape[0]
  T_ref[...] = jnp.concatenate([h, jnp.zeros((np_ - n, F1), f32)], axis=0)
  cdims = (((0,), (1,)), ((), ()))
  el_t = lax.dot_general(Al, h, cdims, preferred_element_type=f32)
  er_t = lax.dot_general(Ar, h, cdims, preferred_element_type=f32)
  elr = jnp.concatenate([el_t, er_t], axis=0)
  elr_ref[...] = jnp.concatenate(
      [elr, jnp.zeros((2 * HEADS, np_ - n), f32)], axis=1)


def _tc_prep_body(doc_ref, word_ref, Wd_ref, bd_ref, Ww_ref, bw_ref,
                  fc_ref, Al_ref, Ar_ref, res_ref,
                  T_ref, elr_ref, resid_ref):
  d = jnp.maximum(jnp.dot(doc_ref[...], Wd_ref[...],
                          preferred_element_type=f32) + bd_ref[...], 0.0)
  w = jnp.maximum(jnp.dot(word_ref[...], Ww_ref[...],
                          preferred_element_type=f32) + bw_ref[...], 0.0)
  feat = jnp.concatenate([d, w], axis=0)
  _store_tables(feat, fc_ref[...], Al_ref[...], Ar_ref[...], T_ref, elr_ref)
  resid_ref[...] = jnp.dot(feat, res_ref[...], preferred_element_type=f32)


def _combine(msg_cat, den, resid, b):
  n = resid.shape[0]
  msg = msg_cat[0, :n] + msg_cat[1, :n]
  dsum = (den[0] + den[1])[:, :n]
  sel = (lax.broadcasted_iota(i32, (HEADS, F1), 1) // OUT_F
         == lax.broadcasted_iota(i32, (HEADS, F1), 0)).astype(f32)
  denf = lax.dot_general(dsum, sel, (((0,), (0,)), ((), ())),
                         preferred_element_type=f32)
  return jnp.maximum(msg / (denf + 1e-9) + resid + b[None, :], 0.0)


def _tc_mid_body(msg_ref, den_ref, resid_ref, b_ref, fc_ref, Al_ref,
                 Ar_ref, T_ref, elr_ref, feat_ref):
  feat = _combine(msg_ref[...], den_ref[...], resid_ref[...], b_ref[...])
  feat_ref[...] = feat
  _store_tables(feat, fc_ref[...], Al_ref[...], Ar_ref[...], T_ref, elr_ref)


def _tc_final_body(msg_ref, den_ref, resid_ref, b_ref, W2_ref,
                   b2_ref, Wc_ref, bc_ref, out_ref):
  feat = _combine(msg_ref[...], den_ref[...], resid_ref[...], b_ref[...])
  g = jnp.max(feat, axis=0, keepdims=True)
  h2 = jnp.maximum(jnp.dot(g, W2_ref[...], preferred_element_type=f32)
                   + b2_ref[...], 0.0)
  out_ref[...] = jnp.dot(h2, Wc_ref[...], preferred_element_type=f32) \
      + bc_ref[...]


def _make_edge_kernel(n_pad, e_real, e0b, ch):
  rows_per_tile = n_pad // NS
  den_rows = n_pad // L
  drows_per_tile = den_rows // NS
  drain = []
  off = 0
  while off < rows_per_tile:
    sz = min(K, rows_per_tile - off)
    drain.append((off, sz))
    off += sz

  mesh = plsc.VectorSubcoreMesh(core_axis_name="c", subcore_axis_name="s")

  @functools.partial(
      pl.kernel,
      out_type=[
          jax.ShapeDtypeStruct((NC, n_pad, OUT_F), f32),
          jax.ShapeDtypeStruct((NC, den_rows, L), f32),
      ],
      mesh=mesh,
      scratch_types=[
          pltpu.VMEM((n_pad, 2), f32),
      ] + [pltpu.VMEM((K,), i32)] * 8
      + [
          pltpu.VMEM((K,), i32),
      ] + [pltpu.VMEM((K, OUT_F), f32)] * 4
      + [
          pltpu.VMEM((K, OUT_F), f32),
          pltpu.VMEM((K, L), f32),
          pltpu.VMEM((K, OUT_F), f32),
          pltpu.VMEM_SHARED((n_pad, OUT_F), f32),
          pltpu.VMEM_SHARED((den_rows, L), f32),
      ] + [pltpu.SemaphoreType.DMA] * 8,
      compiler_params=pltpu.CompilerParams(use_tc_tiling_on_sc=False,
                                           needs_layout_passes=False),
  )
  def edge_kernel(src_hbm, dst_hbm, t_hbm, elr_hbm, msg_hbm, den_hbm,
                  elr_v, s0, s1, s2, s3, d0, d1, d2, d3, dstq_v,
                  r0_, r1_, r2_, r3_, outb_v, den_v,
                  zero_v, acc_sh, accd_sh, g0, g1, g2, g3, i0, i1, i2, i3):
    srcs, dsts = (s0, s1, s2, s3), (d0, d1, d2, d3)
    rows = (r0_, r1_, r2_, r3_)
    gsems, isems = (g0, g1, g2, g3), (i0, i1, i2, i3)
    c = lax.axis_index("c")
    s = lax.axis_index("s")
    wid = c * NS + s
    base0 = wid * (ch * K)
    row0 = s * rows_per_tile
    drow0 = s * drows_per_tile

    pltpu.sync_copy(elr_hbm, elr_v)

    zeros16 = jnp.zeros((L,), f32)

    def zrow(r, _):
      for v in range(OUT_F // L):
        zero_v[r, pl.ds(v * L, L)] = zeros16
      den_v[r, pl.ds(0, L)] = zeros16
      return 0

    lax.fori_loop(0, K, zrow, 0)
    for off_, sz in drain:
      pltpu.sync_copy(zero_v.at[pl.ds(0, sz)],
                      acc_sh.at[pl.ds(row0 + off_, sz)])
    pltpu.sync_copy(den_v.at[pl.ds(0, drows_per_tile)],
                    accd_sh.at[pl.ds(drow0, drows_per_tile)])
    plsc.subcore_barrier()

    iota16 = lax.iota(i32, L)
    czero = jnp.zeros((L,), i32)
    cone = jnp.full((L,), 1, i32)

    def issue_idx(b, kq):
      eb = base0 + kq * K
      pltpu.async_copy(src_hbm.at[pl.ds(eb, K)], srcs[b], isems[b])
      pltpu.async_copy(dst_hbm.at[pl.ds(eb, K)], dsts[b], isems[b])

    def wait_idx(b):
      pltpu.make_async_copy(src_hbm.at[pl.ds(0, K)], srcs[b], isems[b]).wait()
      pltpu.make_async_copy(dst_hbm.at[pl.ds(0, K)], dsts[b], isems[b]).wait()

    def issue_gather(b):
      pltpu.async_copy(t_hbm.at[srcs[b]], rows[b], gsems[b])

    def wait_gather(b):
      pltpu.make_async_copy(t_hbm.at[srcs[b]], rows[b], gsems[b]).wait()

    def compute(b, eb):
      src_v, dst_v, rows_v = srcs[b], dsts[b], rows[b]

      def group_body(g, _):
        r0 = g * L
        src16 = src_v[pl.ds(r0, L)]
        dst16 = dst_v[pl.ds(r0, L)]
        ids = jnp.full((L,), eb, dtype=i32) + r0 + iota16
        keep = (src16 != dst16) | ((ids >= e0b) & (ids < e_real))
        mf = jnp.where(keep, 1.0, 0.0).astype(f32)
        el16 = plsc.load_gather(elr_v, [src16, czero])
        er16 = plsc.load_gather(elr_v, [dst16, cone])
        e16 = el16 + er16
        e16 = jnp.maximum(e16, 0.2 * e16)
        w16 = jnp.exp(e16) * mf
        dstq_v[pl.ds(r0, L)] = lax.shift_right_logical(dst16, 4)
        for i in range(L):
          den_v[r0 + i, pl.ds(0, L)] = zeros16
        plsc.store_scatter(den_v, [r0 + iota16, dst16 & (L - 1)], w16)
        for i in range(L):
          r = r0 + i
          ws = _splat16(w16, i)
          for half in range(OUT_F // L):
            col = half * L
            outb_v[r, pl.ds(col, L)] = ws * rows_v[r, pl.ds(col, L)]
        return 0

      for g in range(K // L):
        group_body(g, 0)
      pltpu.sync_copy(outb_v, acc_sh.at[dst_v], add=True)
      pltpu.sync_copy(den_v, accd_sh.at[dstq_v], add=True)

    for b in range(4):
      issue_idx(b, b)
    for b in range(2):
      wait_idx(b)
      issue_gather(b)

    def super_body(k4, _):
      for b in range(4):
        k = k4 * 4 + b
        wait_idx((b + 2) % 4)
        issue_gather((b + 2) % 4)
        wait_gather(b)
        compute(b, base0 + k * K)
        issue_idx(b, jnp.minimum(k + 4, ch - 1))
      return 0

    lax.fori_loop(0, ch // 4, super_body, 0)
    for b in (0, 1):
      wait_gather(b)
    for b in (2, 3):
      wait_idx(b)
    plsc.subcore_barrier()

    for off_, sz in drain:
      pltpu.sync_copy(acc_sh.at[pl.ds(row0 + off_, sz)],
                      msg_hbm.at[c, pl.ds(row0 + off_, sz)])
    pltpu.sync_copy(accd_sh.at[pl.ds(drow0, drows_per_tile)],
                    den_hbm.at[c, pl.ds(drow0, drows_per_tile)])

  return edge_kernel


def _attn_mat(a):
  flat = a.reshape(F1)
  hd = jnp.arange(F1, dtype=i32)[:, None] // OUT_F
  return jnp.where(hd == jnp.arange(HEADS, dtype=i32)[None, :],
                   flat[:, None], 0.0)


def kernel(doc_feat, word_feat, edge_index, Wd, bd, Ww, bw, fc0, al0, ar0,
           b0, res0, fc1, al1, ar1, b1, W2, b2, Wc, bc):
  nd = doc_feat.shape[0]
  nw = word_feat.shape[0]
  n = nd + nw
  e0 = edge_index.shape[1]
  e0b = 2 * e0
  e_real = e0b + n
  chk = NW * K
  ch = 4 * (-(-e_real // (4 * chk)))
  e_pad = ch * chk
  n_pad = NS * L * (-(-n // (NS * L)))

  ei = edge_index.astype(i32)
  loop = jnp.arange(n, dtype=i32)
  pad = jnp.zeros((e_pad - e_real,), i32)
  src = jnp.concatenate([ei[0], ei[1], loop, pad])
  dst = jnp.concatenate([ei[1], ei[0], loop, pad])

  al0m, ar0m = _attn_mat(al0), _attn_mat(ar0)
  al1m, ar1m = _attn_mat(al1), _attn_mat(ar1)

  table_shapes = [
      jax.ShapeDtypeStruct((n_pad, F1), f32),
      jax.ShapeDtypeStruct((2 * HEADS, n_pad), f32),
  ]

  def run_layer(edge_kernel, T, elr_t):
    msgs, dens = [], []
    for hd in range(HEADS):
      th = lax.slice_in_dim(T, hd * OUT_F, (hd + 1) * OUT_F, axis=1)
      eh = jnp.stack([elr_t[hd], elr_t[HEADS + hd]], axis=1)
      m, d = edge_kernel(src, dst, th, eh)
      msgs.append(m)
      dens.append(d)
    msg_cat = jnp.concatenate(msgs, axis=2)
    den = jnp.stack([d.reshape(NC, n_pad) for d in dens], axis=1)
    return msg_cat, den

  T0, elr0, resid0 = pl.pallas_call(
      _tc_prep_body,
      out_shape=table_shapes + [jax.ShapeDtypeStruct((n, F1), f32)],
  )(doc_feat, word_feat, Wd, bd, Ww, bw, fc0, al0m, ar0m, res0)

  edge_kernel = _make_edge_kernel(n_pad, e_real, e0b, ch)
  msg0, den0 = run_layer(edge_kernel, T0, elr0)

  T1, elr1, feat1 = pl.pallas_call(
      _tc_mid_body,
      out_shape=table_shapes + [jax.ShapeDtypeStruct((n, F1), f32)],
  )(msg0, den0, resid0, b0, fc1, al1m, ar1m)

  msg1, den1 = run_layer(edge_kernel, T1, elr1)

  out = pl.pallas_call(
      _tc_final_body,
      out_shape=jax.ShapeDtypeStruct((1, 2), f32),
  )(msg1, den1, feat1, b1, W2, b2, Wc, bc)

  return out
